# baseline (speedup 1.0000x reference)
import jax
import jax.numpy as jnp
from jax.experimental import pallas as pl
from jax.experimental.pallas import tpu as pltpu

_E = 64
_D = 768
_H = 128
_S = 2048
_A = 6


def _router_body(x_ref, rw_ref, e_ref):
    logits = jnp.dot(x_ref[...], rw_ref[...], preferred_element_type=jnp.float32)
    row = logits[0:1, :]
    mx = jnp.max(row, axis=1, keepdims=True)
    probs = jnp.exp(row - mx)
    pmx = jnp.max(probs, axis=1, keepdims=True)
    ii = jax.lax.broadcasted_iota(jnp.int32, (1, _E), 1)
    idx = jnp.min(jnp.where(probs >= pmx, ii, _E), axis=1)
    e_ref[0] = idx[0]


def _main_body(e_ref, hid_ref, x_ref, df_ref, wemb_ref, bemb_ref, wi_ref,
               wh_ref, bi_ref, bh_ref, wa1_ref, ba1_ref, wa2_ref, ba2_ref,
               wc1_ref, bc1_ref, wc2_ref, bc2_ref,
               nh_ref, pi_ref, val_ref, gi_ref, y_ref):
    e = e_ref[0]

    emb = jnp.maximum(
        jnp.dot(x_ref[...], wemb_ref[0], preferred_element_type=jnp.float32)
        + bemb_ref[pl.ds(e, 1), :],
        0.0,
    )
    col = jax.lax.broadcasted_iota(jnp.int32, (1, 3 * _H), 1)
    scale = jnp.where(col < 2 * _H, 0.5, 1.0)
    gi_ref[...] = (
        jnp.dot(emb, wi_ref[0], preferred_element_type=jnp.float32)
        + bi_ref[pl.ds(e, 1), :]
    ) * scale

    wh = wh_ref[0]
    bh05 = 0.5 * bh_ref[pl.ds(e, 1), :]
    h0 = hid_ref[pl.ds(e, 1), 0, :]

    def step(t, h):
        k = 1.0 - df_ref[0, t]
        c = 0.5 * k
        hu2 = c * h
        hc = jnp.reshape(h, (_H, 1))
        m = jnp.sum(hc * wh, axis=0, keepdims=True)
        gh2 = m * c + bh05
        a = gi_ref[pl.ds(t, 1), :] + gh2
        t_rz = jnp.tanh(a[:, :2 * _H])
        tz = t_rz[:, _H:]
        acc = hu2 * (1.0 + tz)
        wn = 0.5 * (1.0 - tz)
        n = jnp.tanh(a[:, 2 * _H:] + t_rz[:, :_H] * gh2[:, 2 * _H:])
        hn = acc + n * wn
        y_ref[pl.ds(t, 1), :] = hn
        return hn

    hT = jax.lax.fori_loop(0, _S, step, h0, unroll=8)

    y = y_ref[...]
    am = jnp.maximum(
        jnp.dot(y, wa1_ref[0], preferred_element_type=jnp.float32)
        + ba1_ref[pl.ds(e, 1), :],
        0.0,
    )
    pi_ref[...] = (
        jnp.dot(am, wa2_ref[pl.ds(e, 1), :, :][0], preferred_element_type=jnp.float32)
        + ba2_ref[pl.ds(e, 1), :]
    )
    cm = jnp.maximum(
        jnp.dot(y, wc1_ref[0], preferred_element_type=jnp.float32)
        + bc1_ref[pl.ds(e, 1), :],
        0.0,
    )
    val_ref[...] = (
        jnp.dot(cm, wc2_ref[pl.ds(e, 1), :, :][0], preferred_element_type=jnp.float32)
        + bc2_ref[pl.ds(e, 1), :]
    )

    nh_ref[...] = hid_ref[...]
    nh_ref[pl.ds(e, 1), 0, :] = hT


def kernel(hiddens, x, dones, router_W, W_emb, b_emb, Wi, Wh, bi, bh,
           Wa1, ba1, Wa2, ba2, Wc1, bc1, Wc2, bc2):
    e_idx = pl.pallas_call(
        _router_body,
        out_shape=jax.ShapeDtypeStruct((1,), jnp.int32),
        grid=(1,),
        in_specs=[
            pl.BlockSpec((8, _D), lambda i: (0, 0)),
            pl.BlockSpec((_D, _E), lambda i: (0, 0)),
        ],
        out_specs=pl.BlockSpec(memory_space=pltpu.SMEM),
    )(x, router_W)

    df = dones.astype(jnp.float32).reshape(1, _S)

    grid_spec = pltpu.PrefetchScalarGridSpec(
        num_scalar_prefetch=1,
        grid=(1,),
        in_specs=[
            pl.BlockSpec((_E, 1, _H), lambda i, e: (0, 0, 0)),
            pl.BlockSpec((_S, _D), lambda i, e: (0, 0)),
            pl.BlockSpec(memory_space=pltpu.SMEM),
            pl.BlockSpec((1, _D, _H), lambda i, e: (e[0], 0, 0)),
            pl.BlockSpec((_E, _H), lambda i, e: (0, 0)),
            pl.BlockSpec((1, _H, 3 * _H), lambda i, e: (e[0], 0, 0)),
            pl.BlockSpec((1, _H, 3 * _H), lambda i, e: (e[0], 0, 0)),
            pl.BlockSpec((_E, 3 * _H), lambda i, e: (0, 0)),
            pl.BlockSpec((_E, 3 * _H), lambda i, e: (0, 0)),
            pl.BlockSpec((1, _H, _H), lambda i, e: (e[0], 0, 0)),
            pl.BlockSpec((_E, _H), lambda i, e: (0, 0)),
            pl.BlockSpec((_E, _H, _A), lambda i, e: (0, 0, 0)),
            pl.BlockSpec((_E, _A), lambda i, e: (0, 0)),
            pl.BlockSpec((1, _H, _H), lambda i, e: (e[0], 0, 0)),
            pl.BlockSpec((_E, _H), lambda i, e: (0, 0)),
            pl.BlockSpec((_E, _H, 1), lambda i, e: (0, 0, 0)),
            pl.BlockSpec((_E, 1), lambda i, e: (0, 0)),
        ],
        out_specs=[
            pl.BlockSpec((_E, 1, _H), lambda i, e: (0, 0, 0)),
            pl.BlockSpec((_S, _A), lambda i, e: (0, 0)),
            pl.BlockSpec((_S, 1), lambda i, e: (0, 0)),
        ],
        scratch_shapes=[
            pltpu.VMEM((_S, 3 * _H), jnp.float32),
            pltpu.VMEM((_S, _H), jnp.float32),
        ],
    )

    new_hiddens, pi_logits, value2d = pl.pallas_call(
        _main_body,
        grid_spec=grid_spec,
        out_shape=[
            jax.ShapeDtypeStruct((_E, 1, _H), jnp.float32),
            jax.ShapeDtypeStruct((_S, _A), jnp.float32),
            jax.ShapeDtypeStruct((_S, 1), jnp.float32),
        ],
    )(e_idx, hiddens, x, df, W_emb, b_emb, Wi, Wh, bi, bh,
      Wa1, ba1, Wa2, ba2, Wc1, bc1, Wc2, bc2)

    return new_hiddens, pi_logits, value2d[:, 0]


# speedup vs baseline: 1.0421x; 1.0421x over previous
import jax
import jax.numpy as jnp
from jax.experimental import pallas as pl
from jax.experimental.pallas import tpu as pltpu

_E = 64
_D = 768
_H = 128
_S = 2048
_A = 6


def _router_body(x_ref, rw_ref, e_ref):
    logits = jnp.dot(x_ref[...], rw_ref[...], preferred_element_type=jnp.float32)
    row = logits[0:1, :]
    mx = jnp.max(row, axis=1, keepdims=True)
    probs = jnp.exp(row - mx)
    pmx = jnp.max(probs, axis=1, keepdims=True)
    ii = jax.lax.broadcasted_iota(jnp.int32, (1, _E), 1)
    idx = jnp.min(jnp.where(probs >= pmx, ii, _E), axis=1)
    e_ref[0] = idx[0]


def _main_body(e_ref, hid_ref, x_ref, df_ref, wemb_ref, bemb_ref, wi_ref,
               wh_ref, bi_ref, bh_ref, wa1_ref, ba1_ref, wa2_ref, ba2_ref,
               wc1_ref, bc1_ref, wc2_ref, bc2_ref,
               nh_ref, pi_ref, val_ref, gi_ref, y_ref):
    e = e_ref[0]

    emb = jnp.maximum(
        jnp.dot(x_ref[...], wemb_ref[0], preferred_element_type=jnp.float32)
        + bemb_ref[pl.ds(e, 1), :],
        0.0,
    )
    col = jax.lax.broadcasted_iota(jnp.int32, (1, 3 * _H), 1)
    scale = jnp.where(col < 2 * _H, 0.5, 1.0)
    gi_ref[...] = (
        jnp.dot(emb, wi_ref[0], preferred_element_type=jnp.float32)
        + bi_ref[pl.ds(e, 1), :]
    ) * scale

    wh = wh_ref[0]
    bh05 = 0.5 * bh_ref[pl.ds(e, 1), :]
    h0 = hid_ref[pl.ds(e, 1), 0, :]

    def step(t, h):
        k = 1.0 - df_ref[0, t]
        c = 0.5 * k
        hu2 = c * h
        hc = jnp.reshape(h, (_H, 1))
        m = jnp.sum(hc * wh, axis=0, keepdims=True)
        gh2 = m * c + bh05
        a = gi_ref[pl.ds(t, 1), :] + gh2
        t_rz = jnp.tanh(a[:, :2 * _H])
        tz = t_rz[:, _H:]
        acc = hu2 * (1.0 + tz)
        wn = 0.5 * (1.0 - tz)
        n = jnp.tanh(a[:, 2 * _H:] + t_rz[:, :_H] * gh2[:, 2 * _H:])
        hn = acc + n * wn
        y_ref[pl.ds(t, 1), :] = hn
        return hn

    hT = jax.lax.fori_loop(0, _S, step, h0, unroll=8)

    y = y_ref[...]
    am = jnp.maximum(
        jnp.dot(y, wa1_ref[0], preferred_element_type=jnp.float32)
        + ba1_ref[pl.ds(e, 1), :],
        0.0,
    )
    pi_ref[...] = (
        jax.lax.dot_general(am, wa2_ref[0], (((1,), (1,)), ((), ())),
                            preferred_element_type=jnp.float32)
        + ba2_ref[pl.ds(e, 1), :]
    )
    cm = jnp.maximum(
        jnp.dot(y, wc1_ref[0], preferred_element_type=jnp.float32)
        + bc1_ref[pl.ds(e, 1), :],
        0.0,
    )
    val_ref[...] = (
        jnp.dot(cm, wc2_ref[pl.ds(e, 1), :, :][0],
                preferred_element_type=jnp.float32)
        + bc2_ref[e, 0]
    )

    nh_ref[...] = hid_ref[...]
    nh_ref[pl.ds(e, 1), 0, :] = hT


def kernel(hiddens, x, dones, router_W, W_emb, b_emb, Wi, Wh, bi, bh,
           Wa1, ba1, Wa2, ba2, Wc1, bc1, Wc2, bc2):
    e_idx = pl.pallas_call(
        _router_body,
        out_shape=jax.ShapeDtypeStruct((1,), jnp.int32),
        grid=(1,),
        in_specs=[
            pl.BlockSpec((8, _D), lambda i: (0, 0)),
            pl.BlockSpec((_D, _E), lambda i: (0, 0)),
        ],
        out_specs=pl.BlockSpec(memory_space=pltpu.SMEM),
    )(x, router_W)

    df = dones.astype(jnp.float32).reshape(1, _S)

    grid_spec = pltpu.PrefetchScalarGridSpec(
        num_scalar_prefetch=1,
        grid=(1,),
        in_specs=[
            pl.BlockSpec((_E, 1, _H), lambda i, e: (0, 0, 0)),
            pl.BlockSpec((_S, _D), lambda i, e: (0, 0)),
            pl.BlockSpec(memory_space=pltpu.SMEM),
            pl.BlockSpec((1, _D, _H), lambda i, e: (e[0], 0, 0)),
            pl.BlockSpec((_E, _H), lambda i, e: (0, 0)),
            pl.BlockSpec((1, _H, 3 * _H), lambda i, e: (e[0], 0, 0)),
            pl.BlockSpec((1, _H, 3 * _H), lambda i, e: (e[0], 0, 0)),
            pl.BlockSpec((_E, 3 * _H), lambda i, e: (0, 0)),
            pl.BlockSpec((_E, 3 * _H), lambda i, e: (0, 0)),
            pl.BlockSpec((1, _H, _H), lambda i, e: (e[0], 0, 0)),
            pl.BlockSpec((_E, _H), lambda i, e: (0, 0)),
            pl.BlockSpec((1, _A, _H), lambda i, e: (e[0], 0, 0)),
            pl.BlockSpec((_E, _A), lambda i, e: (0, 0)),
            pl.BlockSpec((1, _H, _H), lambda i, e: (e[0], 0, 0)),
            pl.BlockSpec((_E, _H), lambda i, e: (0, 0)),
            pl.BlockSpec((_E, _H, 1), lambda i, e: (0, 0, 0)),
            pl.BlockSpec(memory_space=pltpu.SMEM),
        ],
        out_specs=[
            pl.BlockSpec((_E, 1, _H), lambda i, e: (0, 0, 0)),
            pl.BlockSpec((_S, _A), lambda i, e: (0, 0)),
            pl.BlockSpec((_S, 1), lambda i, e: (0, 0)),
        ],
        scratch_shapes=[
            pltpu.VMEM((_S, 3 * _H), jnp.float32),
            pltpu.VMEM((_S, _H), jnp.float32),
        ],
    )

    new_hiddens, pi_logits, value2d = pl.pallas_call(
        _main_body,
        grid_spec=grid_spec,
        out_shape=[
            jax.ShapeDtypeStruct((_E, 1, _H), jnp.float32),
            jax.ShapeDtypeStruct((_S, _A), jnp.float32),
            jax.ShapeDtypeStruct((_S, 1), jnp.float32),
        ],
    )(e_idx, hiddens, x, df, W_emb, b_emb, Wi, Wh, bi, bh,
      Wa1, ba1, jnp.swapaxes(Wa2, 1, 2), ba2, Wc1, bc1, Wc2, bc2)

    return new_hiddens, pi_logits, value2d[:, 0]
